# baseline (speedup 1.0000x reference)
import functools

import jax
import jax.numpy as jnp
from jax.experimental import pallas as pl
import jax.experimental.pallas.tpu as pltpu

T = 2048
D = 1024
E = 8
FF = 1024
TOPK = 2
TT = 256
NT = T // TT


def _moe_body(x_ref, scale_ref, gk_ref, gb_ref, w1_ref, b1_ref, w2_ref, b2_ref,
              out_ref, normed_s, gw_s, acc_s):
    e = pl.program_id(0)
    t = pl.program_id(1)
    rows = pl.ds(t * TT, TT)

    @pl.when(e == 0)
    def _router():
        xt = x_ref[rows, :]
        rms = jnp.sqrt(jnp.mean(xt * xt, axis=-1, keepdims=True) + 1e-5)
        normed = (xt / rms) * scale_ref[0, :]
        normed_s[rows, :] = normed
        logits = jax.lax.dot_general(
            normed, gk_ref[...], (((1,), (0,)), ((), ())),
            preferred_element_type=jnp.float32) + gb_ref[0, :]
        iota = jax.lax.broadcasted_iota(jnp.int32, (TT, E), 1)
        m1 = jnp.max(logits, axis=-1, keepdims=True)
        idx1 = jnp.min(jnp.where(logits == m1, iota, E), axis=-1, keepdims=True)
        masked = jnp.where(iota == idx1, -jnp.inf, logits)
        m2 = jnp.max(masked, axis=-1, keepdims=True)
        idx2 = jnp.min(jnp.where(masked == m2, iota, E), axis=-1, keepdims=True)
        e2 = jnp.exp(m2 - m1)
        denom = 1.0 + e2
        gw = jnp.where(iota == idx1, 1.0 / denom, 0.0) + \
             jnp.where(iota == idx2, e2 / denom, 0.0)
        gw_s[rows, :] = gw

    normed = normed_s[rows, :]
    w1 = w1_ref[0]
    m1out = jax.lax.dot_general(
        normed, w1, (((1,), (1,)), ((), ())),
        preferred_element_type=jnp.float32) + b1_ref[0, 0, :]
    gate_part = jnp.minimum(m1out[:, :FF], 7.0)
    linear_part = jnp.clip(m1out[:, FF:], -7.0, 7.0)
    swish_gate = gate_part * jax.nn.sigmoid(1.702 * gate_part)
    activated = swish_gate * (linear_part + 1.0)
    w2 = w2_ref[0]
    eout = jax.lax.dot_general(
        activated, w2, (((1,), (1,)), ((), ())),
        preferred_element_type=jnp.float32) + b2_ref[0, 0, :]
    gwt = gw_s[rows, :]
    eiota = jax.lax.broadcasted_iota(jnp.int32, (TT, E), 1)
    gwe = jnp.sum(jnp.where(eiota == e, gwt, 0.0), axis=-1, keepdims=True)
    contrib = eout * gwe

    @pl.when(e == 0)
    def _init():
        acc_s[rows, :] = x_ref[rows, :] + contrib

    @pl.when(e > 0)
    def _acc():
        acc_s[rows, :] = acc_s[rows, :] + contrib

    @pl.when(e == E - 1)
    def _emit():
        out_ref[...] = acc_s[rows, :]


@jax.jit
def kernel(x, scale, gate_kernel, gate_bias, mlp1_weight, mlp1_bias,
           mlp2_weight, mlp2_bias):
    grid = (E, NT)
    out = pl.pallas_call(
        _moe_body,
        grid=grid,
        in_specs=[
            pl.BlockSpec((T, D), lambda e, t: (0, 0)),
            pl.BlockSpec((1, D), lambda e, t: (0, 0)),
            pl.BlockSpec((D, E), lambda e, t: (0, 0)),
            pl.BlockSpec((1, E), lambda e, t: (0, 0)),
            pl.BlockSpec((1, 2 * FF, D), lambda e, t: (e, 0, 0)),
            pl.BlockSpec((1, 1, 2 * FF), lambda e, t: (e, 0, 0)),
            pl.BlockSpec((1, D, FF), lambda e, t: (e, 0, 0)),
            pl.BlockSpec((1, 1, D), lambda e, t: (e, 0, 0)),
        ],
        out_specs=pl.BlockSpec((TT, D), lambda e, t: (t, 0)),
        out_shape=jax.ShapeDtypeStruct((T, D), jnp.float32),
        scratch_shapes=[
            pltpu.VMEM((T, D), jnp.float32),
            pltpu.VMEM((T, E), jnp.float32),
            pltpu.VMEM((T, D), jnp.float32),
        ],
    )(x, scale.reshape(1, D), gate_kernel, gate_bias.reshape(1, E),
      mlp1_weight, mlp1_bias.reshape(E, 1, 2 * FF),
      mlp2_weight, mlp2_bias.reshape(E, 1, D))
    return out
